# Initial kernel scaffold; baseline (speedup 1.0000x reference)
import functools

import jax
import jax.numpy as jnp
from jax import lax
from jax.experimental import pallas as pl
from jax.experimental.pallas import tpu as pltpu
from jax.experimental.pallas import tpu_sc as plsc

N = 10000
D = 128
NP = 10240
NC, NS = 2, 16
NW = NC * NS
CH = 128
R = 1024


def _sc_mesh():
    return plsc.VectorSubcoreMesh(
        core_axis_name="c", subcore_axis_name="s", num_cores=NC, num_subcores=NS
    )


def _sc_scatter_rows(h, src3, dst3, zeros_rows, n_ch):
    rpt = NP // NS

    @functools.partial(
        pl.kernel,
        out_type=jax.ShapeDtypeStruct((NC, NP, D), jnp.float32),
        mesh=_sc_mesh(),
        scratch_types=[
            pltpu.VMEM((n_ch, CH), jnp.int32),
            pltpu.VMEM((n_ch, CH), jnp.int32),
            pltpu.VMEM((CH, D), jnp.float32),
            pltpu.VMEM_SHARED((NP, D), jnp.float32),
            pltpu.SemaphoreType.DMA,
        ],
    )
    def k(h_hbm, src_hbm, dst_hbm, z_hbm, out_hbm, src_v, dst_v, rows_v, acc, sem):
        cid = lax.axis_index("c")
        sid = lax.axis_index("s")
        wid = cid * NS + sid
        pltpu.sync_copy(z_hbm.at[pl.ds(sid * rpt, rpt)], acc.at[pl.ds(sid * rpt, rpt)])
        pltpu.sync_copy(src_hbm.at[wid], src_v)
        pltpu.sync_copy(dst_hbm.at[wid], dst_v)
        plsc.subcore_barrier()

        def body(j, carry):
            pltpu.async_copy(h_hbm.at[src_v.at[j]], rows_v, sem).wait()
            pltpu.sync_copy(rows_v, acc.at[dst_v.at[j]], add=True)
            return carry

        lax.fori_loop(0, n_ch, body, 0)
        plsc.subcore_barrier()
        pltpu.sync_copy(
            acc.at[pl.ds(sid * rpt, rpt)], out_hbm.at[cid, pl.ds(sid * rpt, rpt)]
        )

    return k(h, src3, dst3, zeros_rows)


def _sc_degree(dst3, ones_rows, zeros_rows16, n_ch):
    rpt = NP // NS

    @functools.partial(
        pl.kernel,
        out_type=jax.ShapeDtypeStruct((NC, NP, 16), jnp.float32),
        mesh=_sc_mesh(),
        scratch_types=[
            pltpu.VMEM((n_ch, CH), jnp.int32),
            pltpu.VMEM((CH, 16), jnp.float32),
            pltpu.VMEM_SHARED((NP, 16), jnp.float32),
        ],
    )
    def k(dst_hbm, ones_hbm, z_hbm, out_hbm, dst_v, ones_v, acc):
        cid = lax.axis_index("c")
        sid = lax.axis_index("s")
        wid = cid * NS + sid
        pltpu.sync_copy(z_hbm.at[pl.ds(sid * rpt, rpt)], acc.at[pl.ds(sid * rpt, rpt)])
        pltpu.sync_copy(dst_hbm.at[wid], dst_v)
        pltpu.sync_copy(ones_hbm, ones_v)
        plsc.subcore_barrier()

        def body(j, carry):
            pltpu.sync_copy(ones_v, acc.at[dst_v.at[j]], add=True)
            return carry

        lax.fori_loop(0, n_ch, body, 0)
        plsc.subcore_barrier()
        pltpu.sync_copy(
            acc.at[pl.ds(sid * rpt, rpt)], out_hbm.at[cid, pl.ds(sid * rpt, rpt)]
        )

    return k(dst3, ones_rows, zeros_rows16)


def _tc_matmul(x, W):
    def body(x_ref, w_ref, o_ref):
        o_ref[...] = jnp.dot(x_ref[...], w_ref[...], preferred_element_type=jnp.float32)

    return pl.pallas_call(
        body,
        grid=(NP // R,),
        in_specs=[
            pl.BlockSpec((R, D), lambda i: (i, 0)),
            pl.BlockSpec((D, D), lambda i: (0, 0)),
        ],
        out_specs=pl.BlockSpec((R, D), lambda i: (i, 0)),
        out_shape=jax.ShapeDtypeStruct((NP, D), jnp.float32),
    )(x, W)


def _tc_scale(deg_parts, h_raw):
    def body(deg_ref, h_ref, h1p_ref, dinv_ref):
        d = deg_ref[...]
        degsum = d[0, :, 0:1] + d[1, :, 0:1] + 1.0
        dinv = lax.rsqrt(degsum)
        dinvb = jnp.broadcast_to(dinv, (R, D))
        dinv_ref[...] = dinvb
        h1p_ref[...] = h_ref[...] * dinvb

    return pl.pallas_call(
        body,
        grid=(NP // R,),
        in_specs=[
            pl.BlockSpec((NC, R, 16), lambda i: (0, i, 0)),
            pl.BlockSpec((R, D), lambda i: (i, 0)),
        ],
        out_specs=[
            pl.BlockSpec((R, D), lambda i: (i, 0)),
            pl.BlockSpec((R, D), lambda i: (i, 0)),
        ],
        out_shape=[
            jax.ShapeDtypeStruct((NP, D), jnp.float32),
            jax.ShapeDtypeStruct((NP, D), jnp.float32),
        ],
    )(deg_parts, h_raw)


def _tc_combine(s_parts, hp, dinvb, b2d, relu):
    def body(s_ref, hp_ref, dinv_ref, b_ref, o_ref):
        t = (s_ref[0] + s_ref[1] + hp_ref[...]) * dinv_ref[...] + b_ref[...]
        o_ref[...] = jnp.maximum(t, 0.0) if relu else t

    return pl.pallas_call(
        body,
        grid=(NP // R,),
        in_specs=[
            pl.BlockSpec((NC, R, D), lambda i: (0, i, 0)),
            pl.BlockSpec((R, D), lambda i: (i, 0)),
            pl.BlockSpec((R, D), lambda i: (i, 0)),
            pl.BlockSpec((1, D), lambda i: (0, 0)),
        ],
        out_specs=pl.BlockSpec((R, D), lambda i: (i, 0)),
        out_shape=jax.ShapeDtypeStruct((NP, D), jnp.float32),
    )(s_parts, hp, dinvb, b2d)


def _tc_matmul_scale(x, W, dinvb):
    def body(x_ref, w_ref, dinv_ref, o_ref):
        o_ref[...] = (
            jnp.dot(x_ref[...], w_ref[...], preferred_element_type=jnp.float32)
            * dinv_ref[...]
        )

    return pl.pallas_call(
        body,
        grid=(NP // R,),
        in_specs=[
            pl.BlockSpec((R, D), lambda i: (i, 0)),
            pl.BlockSpec((D, D), lambda i: (0, 0)),
            pl.BlockSpec((R, D), lambda i: (i, 0)),
        ],
        out_specs=pl.BlockSpec((R, D), lambda i: (i, 0)),
        out_shape=jax.ShapeDtypeStruct((NP, D), jnp.float32),
    )(x, W, dinvb)


def kernel(x, edge_index, W1, b1, W2, b2):
    E = edge_index.shape[1]
    n_ch = -(-E // (NW * CH))
    e_pad = NW * n_ch * CH

    pad = jnp.full((e_pad - E,), N, dtype=edge_index.dtype)
    src3 = jnp.concatenate([edge_index[0], pad]).reshape(NW, n_ch, CH)
    dst3 = jnp.concatenate([edge_index[1], pad]).reshape(NW, n_ch, CH)

    xp = jnp.pad(x, ((0, NP - N), (0, 0)))
    zeros_rows = jnp.zeros((NP, D), jnp.float32)
    zeros16 = jnp.zeros((NP, 16), jnp.float32)
    ones_rows = jnp.ones((CH, 16), jnp.float32)
    b1_2d = b1.reshape(1, D)
    b2_2d = b2.reshape(1, D)

    deg_parts = _sc_degree(dst3, ones_rows, zeros16, n_ch)
    h_raw = _tc_matmul(xp, W1)
    h1p, dinvb = _tc_scale(deg_parts, h_raw)

    s1 = _sc_scatter_rows(h1p, src3, dst3, zeros_rows, n_ch)
    h1 = _tc_combine(s1, h1p, dinvb, b1_2d, relu=True)

    h2p = _tc_matmul_scale(h1, W2, dinvb)
    s2 = _sc_scatter_rows(h2p, src3, dst3, zeros_rows, n_ch)
    out = _tc_combine(s2, h2p, dinvb, b2_2d, relu=False)

    return out[:N]


# speedup vs baseline: 14.5922x; 14.5922x over previous
import functools

import jax
import jax.numpy as jnp
from jax import lax
from jax.experimental import pallas as pl
from jax.experimental.pallas import tpu as pltpu
from jax.experimental.pallas import tpu_sc as plsc

N = 10000
D = 128
NP = 10240
NC, NS = 2, 16
NW = NC * NS
CH = 128
R = 1024


def _sc_mesh():
    return plsc.VectorSubcoreMesh(
        core_axis_name="c", subcore_axis_name="s", num_cores=NC, num_subcores=NS
    )


def _sc_scatter_rows(h, src3, dst3, zeros_rows, n_ch):
    rpt = NP // NS

    @functools.partial(
        pl.kernel,
        out_type=jax.ShapeDtypeStruct((NC, NP, D), jnp.float32),
        mesh=_sc_mesh(),
        scratch_types=[
            pltpu.VMEM((CH,), jnp.int32),
            pltpu.VMEM((CH,), jnp.int32),
            pltpu.VMEM((CH, D), jnp.float32),
            pltpu.VMEM_SHARED((NP, D), jnp.float32),
            pltpu.SemaphoreType.DMA,
        ],
    )
    def k(h_hbm, src_hbm, dst_hbm, z_hbm, out_hbm, src_v, dst_v, rows_v, acc, sem):
        cid = lax.axis_index("c")
        sid = lax.axis_index("s")
        wid = cid * NS + sid
        pltpu.sync_copy(z_hbm.at[pl.ds(sid * rpt, rpt)], acc.at[pl.ds(sid * rpt, rpt)])
        plsc.subcore_barrier()

        def body(j, carry):
            pltpu.sync_copy(src_hbm.at[wid, j], src_v)
            pltpu.sync_copy(dst_hbm.at[wid, j], dst_v)
            pltpu.async_copy(h_hbm.at[src_v], rows_v, sem).wait()
            pltpu.sync_copy(rows_v, acc.at[dst_v], add=True)
            return carry

        lax.fori_loop(0, n_ch, body, 0)
        plsc.subcore_barrier()
        pltpu.sync_copy(
            acc.at[pl.ds(sid * rpt, rpt)], out_hbm.at[cid, pl.ds(sid * rpt, rpt)]
        )

    return k(h, src3, dst3, zeros_rows)


def _sc_degree(dst3, ones_rows, zeros_rows, n_ch):
    rpt = NP // NS

    @functools.partial(
        pl.kernel,
        out_type=jax.ShapeDtypeStruct((NC, NP, D), jnp.float32),
        mesh=_sc_mesh(),
        scratch_types=[
            pltpu.VMEM((CH,), jnp.int32),
            pltpu.VMEM((CH, D), jnp.float32),
            pltpu.VMEM_SHARED((NP, D), jnp.float32),
        ],
    )
    def k(dst_hbm, ones_hbm, z_hbm, out_hbm, dst_v, ones_v, acc):
        cid = lax.axis_index("c")
        sid = lax.axis_index("s")
        wid = cid * NS + sid
        pltpu.sync_copy(z_hbm.at[pl.ds(sid * rpt, rpt)], acc.at[pl.ds(sid * rpt, rpt)])
        pltpu.sync_copy(ones_hbm, ones_v)
        plsc.subcore_barrier()

        def body(j, carry):
            pltpu.sync_copy(dst_hbm.at[wid, j], dst_v)
            pltpu.sync_copy(ones_v, acc.at[dst_v], add=True)
            return carry

        lax.fori_loop(0, n_ch, body, 0)
        plsc.subcore_barrier()
        pltpu.sync_copy(
            acc.at[pl.ds(sid * rpt, rpt)], out_hbm.at[cid, pl.ds(sid * rpt, rpt)]
        )

    return k(dst3, ones_rows, zeros_rows)


def _tc_matmul(x, W):
    def body(x_ref, w_ref, o_ref):
        o_ref[...] = jnp.dot(x_ref[...], w_ref[...], preferred_element_type=jnp.float32)

    return pl.pallas_call(
        body,
        grid=(NP // R,),
        in_specs=[
            pl.BlockSpec((R, D), lambda i: (i, 0)),
            pl.BlockSpec((D, D), lambda i: (0, 0)),
        ],
        out_specs=pl.BlockSpec((R, D), lambda i: (i, 0)),
        out_shape=jax.ShapeDtypeStruct((NP, D), jnp.float32),
    )(x, W)


def _tc_scale(deg_parts, h_raw):
    def body(deg_ref, h_ref, h1p_ref, dinv_ref):
        d = deg_ref[...]
        degsum = d[0, :, 0:1] + d[1, :, 0:1] + 1.0
        dinv = lax.rsqrt(degsum)
        dinvb = jnp.broadcast_to(dinv, (R, D))
        dinv_ref[...] = dinvb
        h1p_ref[...] = h_ref[...] * dinvb

    return pl.pallas_call(
        body,
        grid=(NP // R,),
        in_specs=[
            pl.BlockSpec((NC, R, D), lambda i: (0, i, 0)),
            pl.BlockSpec((R, D), lambda i: (i, 0)),
        ],
        out_specs=[
            pl.BlockSpec((R, D), lambda i: (i, 0)),
            pl.BlockSpec((R, D), lambda i: (i, 0)),
        ],
        out_shape=[
            jax.ShapeDtypeStruct((NP, D), jnp.float32),
            jax.ShapeDtypeStruct((NP, D), jnp.float32),
        ],
    )(deg_parts, h_raw)


def _tc_combine(s_parts, hp, dinvb, b2d, relu):
    def body(s_ref, hp_ref, dinv_ref, b_ref, o_ref):
        t = (s_ref[0] + s_ref[1] + hp_ref[...]) * dinv_ref[...] + b_ref[...]
        o_ref[...] = jnp.maximum(t, 0.0) if relu else t

    return pl.pallas_call(
        body,
        grid=(NP // R,),
        in_specs=[
            pl.BlockSpec((NC, R, D), lambda i: (0, i, 0)),
            pl.BlockSpec((R, D), lambda i: (i, 0)),
            pl.BlockSpec((R, D), lambda i: (i, 0)),
            pl.BlockSpec((1, D), lambda i: (0, 0)),
        ],
        out_specs=pl.BlockSpec((R, D), lambda i: (i, 0)),
        out_shape=jax.ShapeDtypeStruct((NP, D), jnp.float32),
    )(s_parts, hp, dinvb, b2d)


def _tc_matmul_scale(x, W, dinvb):
    def body(x_ref, w_ref, dinv_ref, o_ref):
        o_ref[...] = (
            jnp.dot(x_ref[...], w_ref[...], preferred_element_type=jnp.float32)
            * dinv_ref[...]
        )

    return pl.pallas_call(
        body,
        grid=(NP // R,),
        in_specs=[
            pl.BlockSpec((R, D), lambda i: (i, 0)),
            pl.BlockSpec((D, D), lambda i: (0, 0)),
            pl.BlockSpec((R, D), lambda i: (i, 0)),
        ],
        out_specs=pl.BlockSpec((R, D), lambda i: (i, 0)),
        out_shape=jax.ShapeDtypeStruct((NP, D), jnp.float32),
    )(x, W, dinvb)


def kernel(x, edge_index, W1, b1, W2, b2):
    E = edge_index.shape[1]
    n_ch = -(-E // (NW * CH))
    e_pad = NW * n_ch * CH

    pad = N + jnp.arange(e_pad - E, dtype=edge_index.dtype) % (NP - N)
    src3 = jnp.concatenate([edge_index[0], pad]).reshape(NW, n_ch, CH)
    dst3 = jnp.concatenate([edge_index[1], pad]).reshape(NW, n_ch, CH)

    xp = jnp.pad(x, ((0, NP - N), (0, 0)))
    zeros_rows = jnp.zeros((NP, D), jnp.float32)
    ones_rows = jnp.ones((CH, D), jnp.float32)
    b1_2d = b1.reshape(1, D)
    b2_2d = b2.reshape(1, D)

    deg_parts = _sc_degree(dst3, ones_rows, zeros_rows, n_ch)
    h_raw = _tc_matmul(xp, W1)
    h1p, dinvb = _tc_scale(deg_parts, h_raw)

    s1 = _sc_scatter_rows(h1p, src3, dst3, zeros_rows, n_ch)
    h1 = _tc_combine(s1, h1p, dinvb, b1_2d, relu=True)

    h2p = _tc_matmul_scale(h1, W2, dinvb)
    s2 = _sc_scatter_rows(h2p, src3, dst3, zeros_rows, n_ch)
    out = _tc_combine(s2, h2p, dinvb, b2_2d, relu=False)

    return out[:N]


# speedup vs baseline: 26.8437x; 1.8396x over previous
import functools

import jax
import jax.numpy as jnp
from jax import lax
from jax.experimental import pallas as pl
from jax.experimental.pallas import tpu as pltpu
from jax.experimental.pallas import tpu_sc as plsc

N = 10000
D = 128
NP = 10240
NC, NS = 2, 16
NW = NC * NS
CH = 128
R = 1024


def _sc_mesh():
    return plsc.VectorSubcoreMesh(
        core_axis_name="c", subcore_axis_name="s", num_cores=NC, num_subcores=NS
    )


def _sc_scatter_rows(h, src3, dst3, zeros_rows, n_ch):
    rpt = NP // NS

    n_g = (n_ch + 1) // 2

    @functools.partial(
        pl.kernel,
        out_type=jax.ShapeDtypeStruct((NC, NP, D), jnp.float32),
        mesh=_sc_mesh(),
        scratch_types=[
            pltpu.VMEM((n_ch, CH), jnp.int32),
            pltpu.VMEM((CH,), jnp.int32),
            pltpu.VMEM((CH,), jnp.int32),
            pltpu.VMEM((CH, D), jnp.float32),
            pltpu.VMEM((CH, D), jnp.float32),
            pltpu.VMEM_SHARED((NP, D), jnp.float32),
            pltpu.SemaphoreType.DMA,
            pltpu.SemaphoreType.DMA,
        ],
    )
    def k(h_hbm, src_hbm, dst_hbm, z_hbm, out_hbm, src_v, d0, d1, r0, r1, acc,
          sem0, sem1):
        cid = lax.axis_index("c")
        sid = lax.axis_index("s")
        wid = cid * NS + sid
        rows = (r0, r1)
        dsts = (d0, d1)
        sems = (sem0, sem1)
        pltpu.sync_copy(src_hbm.at[wid], src_v)
        pltpu.sync_copy(z_hbm.at[pl.ds(sid * rpt, rpt)], acc.at[pl.ds(sid * rpt, rpt)])
        plsc.subcore_barrier()

        pltpu.async_copy(h_hbm.at[src_v.at[0]], rows[0], sems[0])
        pltpu.async_copy(dst_hbm.at[wid, 0], dsts[0], sems[0])

        def body(g, carry):
            for b in range(2):
                j = g * 2 + b
                nxt = j + 1

                @pl.when(nxt < n_ch)
                def _():
                    pltpu.async_copy(
                        h_hbm.at[src_v.at[nxt]], rows[1 - b], sems[1 - b]
                    )
                    pltpu.async_copy(dst_hbm.at[wid, nxt], dsts[1 - b], sems[1 - b])

                @pl.when(j < n_ch)
                def _():
                    pltpu.make_async_copy(
                        h_hbm.at[src_v.at[j]], rows[b], sems[b]
                    ).wait()
                    pltpu.make_async_copy(
                        dst_hbm.at[wid, j], dsts[b], sems[b]
                    ).wait()
                    pltpu.sync_copy(rows[b], acc.at[dsts[b]], add=True)

            return carry

        lax.fori_loop(0, n_g, body, 0)
        plsc.subcore_barrier()
        pltpu.sync_copy(
            acc.at[pl.ds(sid * rpt, rpt)], out_hbm.at[cid, pl.ds(sid * rpt, rpt)]
        )

    return k(h, src3, dst3, zeros_rows)


def _sc_degree(dst3, ones_rows, zeros_rows, n_ch):
    rpt = NP // NS

    @functools.partial(
        pl.kernel,
        out_type=jax.ShapeDtypeStruct((NC, NP, D), jnp.float32),
        mesh=_sc_mesh(),
        scratch_types=[
            pltpu.VMEM((n_ch, CH), jnp.int32),
            pltpu.VMEM((CH, D), jnp.float32),
            pltpu.VMEM_SHARED((NP, D), jnp.float32),
        ],
    )
    def k(dst_hbm, ones_hbm, z_hbm, out_hbm, dst_v, ones_v, acc):
        cid = lax.axis_index("c")
        sid = lax.axis_index("s")
        wid = cid * NS + sid
        pltpu.sync_copy(z_hbm.at[pl.ds(sid * rpt, rpt)], acc.at[pl.ds(sid * rpt, rpt)])
        pltpu.sync_copy(dst_hbm.at[wid], dst_v)
        pltpu.sync_copy(ones_hbm, ones_v)
        plsc.subcore_barrier()

        def body(j, carry):
            pltpu.sync_copy(ones_v, acc.at[dst_v.at[j]], add=True)
            return carry

        lax.fori_loop(0, n_ch, body, 0)
        plsc.subcore_barrier()
        pltpu.sync_copy(
            acc.at[pl.ds(sid * rpt, rpt)], out_hbm.at[cid, pl.ds(sid * rpt, rpt)]
        )

    return k(dst3, ones_rows, zeros_rows)


def _tc_matmul(x, W):
    def body(x_ref, w_ref, o_ref):
        o_ref[...] = jnp.dot(x_ref[...], w_ref[...], preferred_element_type=jnp.float32)

    return pl.pallas_call(
        body,
        grid=(NP // R,),
        in_specs=[
            pl.BlockSpec((R, D), lambda i: (i, 0)),
            pl.BlockSpec((D, D), lambda i: (0, 0)),
        ],
        out_specs=pl.BlockSpec((R, D), lambda i: (i, 0)),
        out_shape=jax.ShapeDtypeStruct((NP, D), jnp.float32),
    )(x, W)


def _tc_scale(deg_parts, h_raw):
    def body(deg_ref, h_ref, h1p_ref, dinv_ref):
        d = deg_ref[...]
        degsum = d[0, :, 0:1] + d[1, :, 0:1] + 1.0
        dinv = lax.rsqrt(degsum)
        dinvb = jnp.broadcast_to(dinv, (R, D))
        dinv_ref[...] = dinvb
        h1p_ref[...] = h_ref[...] * dinvb

    return pl.pallas_call(
        body,
        grid=(NP // R,),
        in_specs=[
            pl.BlockSpec((NC, R, D), lambda i: (0, i, 0)),
            pl.BlockSpec((R, D), lambda i: (i, 0)),
        ],
        out_specs=[
            pl.BlockSpec((R, D), lambda i: (i, 0)),
            pl.BlockSpec((R, D), lambda i: (i, 0)),
        ],
        out_shape=[
            jax.ShapeDtypeStruct((NP, D), jnp.float32),
            jax.ShapeDtypeStruct((NP, D), jnp.float32),
        ],
    )(deg_parts, h_raw)


def _tc_combine(s_parts, hp, dinvb, b2d, relu):
    def body(s_ref, hp_ref, dinv_ref, b_ref, o_ref):
        t = (s_ref[0] + s_ref[1] + hp_ref[...]) * dinv_ref[...] + b_ref[...]
        o_ref[...] = jnp.maximum(t, 0.0) if relu else t

    return pl.pallas_call(
        body,
        grid=(NP // R,),
        in_specs=[
            pl.BlockSpec((NC, R, D), lambda i: (0, i, 0)),
            pl.BlockSpec((R, D), lambda i: (i, 0)),
            pl.BlockSpec((R, D), lambda i: (i, 0)),
            pl.BlockSpec((1, D), lambda i: (0, 0)),
        ],
        out_specs=pl.BlockSpec((R, D), lambda i: (i, 0)),
        out_shape=jax.ShapeDtypeStruct((NP, D), jnp.float32),
    )(s_parts, hp, dinvb, b2d)


def _tc_matmul_scale(x, W, dinvb):
    def body(x_ref, w_ref, dinv_ref, o_ref):
        o_ref[...] = (
            jnp.dot(x_ref[...], w_ref[...], preferred_element_type=jnp.float32)
            * dinv_ref[...]
        )

    return pl.pallas_call(
        body,
        grid=(NP // R,),
        in_specs=[
            pl.BlockSpec((R, D), lambda i: (i, 0)),
            pl.BlockSpec((D, D), lambda i: (0, 0)),
            pl.BlockSpec((R, D), lambda i: (i, 0)),
        ],
        out_specs=pl.BlockSpec((R, D), lambda i: (i, 0)),
        out_shape=jax.ShapeDtypeStruct((NP, D), jnp.float32),
    )(x, W, dinvb)


def kernel(x, edge_index, W1, b1, W2, b2):
    E = edge_index.shape[1]
    n_ch = -(-E // (NW * CH))
    e_pad = NW * n_ch * CH

    pad = N + jnp.arange(e_pad - E, dtype=edge_index.dtype) % (NP - N)
    src3 = jnp.concatenate([edge_index[0], pad]).reshape(NW, n_ch, CH)
    dst3 = jnp.concatenate([edge_index[1], pad]).reshape(NW, n_ch, CH)

    xp = jnp.pad(x, ((0, NP - N), (0, 0)))
    zeros_rows = jnp.zeros((NP, D), jnp.float32)
    ones_rows = jnp.ones((CH, D), jnp.float32)
    b1_2d = b1.reshape(1, D)
    b2_2d = b2.reshape(1, D)

    deg_parts = _sc_degree(dst3, ones_rows, zeros_rows, n_ch)
    h_raw = _tc_matmul(xp, W1)
    h1p, dinvb = _tc_scale(deg_parts, h_raw)

    s1 = _sc_scatter_rows(h1p, src3, dst3, zeros_rows, n_ch)
    h1 = _tc_combine(s1, h1p, dinvb, b1_2d, relu=True)

    h2p = _tc_matmul_scale(h1, W2, dinvb)
    s2 = _sc_scatter_rows(h2p, src3, dst3, zeros_rows, n_ch)
    out = _tc_combine(s2, h2p, dinvb, b2_2d, relu=False)

    return out[:N]


# speedup vs baseline: 27.5641x; 1.0268x over previous
import functools

import jax
import jax.numpy as jnp
from jax import lax
from jax.experimental import pallas as pl
from jax.experimental.pallas import tpu as pltpu
from jax.experimental.pallas import tpu_sc as plsc

N = 10000
D = 128
NP = 10240
NC, NS = 2, 16
NW = NC * NS
CH = 128
R = 1024
DW = 128


def _sc_mesh():
    return plsc.VectorSubcoreMesh(
        core_axis_name="c", subcore_axis_name="s", num_cores=NC, num_subcores=NS
    )


def _sc_scatter_rows(h, src3, dst3, zeros_rows, n_ch):
    rpt = NP // NS

    n_g = (n_ch + 1) // 2

    @functools.partial(
        pl.kernel,
        out_type=jax.ShapeDtypeStruct((NC, NP, D), jnp.float32),
        mesh=_sc_mesh(),
        scratch_types=[
            pltpu.VMEM((n_ch, CH), jnp.int32),
            pltpu.VMEM((CH,), jnp.int32),
            pltpu.VMEM((CH,), jnp.int32),
            pltpu.VMEM((CH, D), jnp.float32),
            pltpu.VMEM((CH, D), jnp.float32),
            pltpu.VMEM_SHARED((NP, D), jnp.float32),
            pltpu.SemaphoreType.DMA,
            pltpu.SemaphoreType.DMA,
        ],
    )
    def k(h_hbm, src_hbm, dst_hbm, z_hbm, out_hbm, src_v, d0, d1, r0, r1, acc,
          sem0, sem1):
        cid = lax.axis_index("c")
        sid = lax.axis_index("s")
        wid = cid * NS + sid
        rows = (r0, r1)
        dsts = (d0, d1)
        sems = (sem0, sem1)
        pltpu.sync_copy(src_hbm.at[wid], src_v)
        pltpu.sync_copy(z_hbm.at[pl.ds(sid * rpt, rpt)], acc.at[pl.ds(sid * rpt, rpt)])
        plsc.subcore_barrier()

        pltpu.async_copy(h_hbm.at[src_v.at[0]], rows[0], sems[0])
        pltpu.async_copy(dst_hbm.at[wid, 0], dsts[0], sems[0])

        def body(g, carry):
            for b in range(2):
                j = g * 2 + b
                nxt = j + 1

                @pl.when(nxt < n_ch)
                def _():
                    pltpu.async_copy(
                        h_hbm.at[src_v.at[nxt]], rows[1 - b], sems[1 - b]
                    )
                    pltpu.async_copy(dst_hbm.at[wid, nxt], dsts[1 - b], sems[1 - b])

                @pl.when(j < n_ch)
                def _():
                    pltpu.make_async_copy(
                        h_hbm.at[src_v.at[j]], rows[b], sems[b]
                    ).wait()
                    pltpu.make_async_copy(
                        dst_hbm.at[wid, j], dsts[b], sems[b]
                    ).wait()
                    pltpu.sync_copy(rows[b], acc.at[dsts[b]], add=True)

            return carry

        lax.fori_loop(0, n_g, body, 0)
        plsc.subcore_barrier()
        pltpu.sync_copy(
            acc.at[pl.ds(sid * rpt, rpt)], out_hbm.at[cid, pl.ds(sid * rpt, rpt)]
        )

    return k(h, src3, dst3, zeros_rows)


def _sc_degree(dst3, ones_rows, zeros_rows, n_ch):
    rpt = NP // NS

    @functools.partial(
        pl.kernel,
        out_type=jax.ShapeDtypeStruct((NC, NP, DW), jnp.float32),
        mesh=_sc_mesh(),
        scratch_types=[
            pltpu.VMEM((n_ch, CH), jnp.int32),
            pltpu.VMEM((CH, DW), jnp.float32),
            pltpu.VMEM_SHARED((NP, DW), jnp.float32),
        ],
    )
    def k(dst_hbm, ones_hbm, z_hbm, out_hbm, dst_v, ones_v, acc):
        cid = lax.axis_index("c")
        sid = lax.axis_index("s")
        wid = cid * NS + sid
        pltpu.sync_copy(z_hbm.at[pl.ds(sid * rpt, rpt)], acc.at[pl.ds(sid * rpt, rpt)])
        pltpu.sync_copy(dst_hbm.at[wid], dst_v)
        pltpu.sync_copy(ones_hbm, ones_v)
        plsc.subcore_barrier()

        def body(j, carry):
            pltpu.sync_copy(ones_v, acc.at[dst_v.at[j]], add=True)
            return carry

        lax.fori_loop(0, n_ch, body, 0)
        plsc.subcore_barrier()
        pltpu.sync_copy(
            acc.at[pl.ds(sid * rpt, rpt)], out_hbm.at[cid, pl.ds(sid * rpt, rpt)]
        )

    return k(dst3, ones_rows, zeros_rows)


def _tc_mm1_scale(x, W, deg_parts):
    def body(x_ref, w_ref, deg_ref, h1p_ref, dinv_ref):
        d = deg_ref[...]
        degsum = d[0, :, 0:1] + d[1, :, 0:1] + 1.0
        dinvb = jnp.broadcast_to(lax.rsqrt(degsum), (R, D))
        dinv_ref[...] = dinvb
        h1p_ref[...] = (
            jnp.dot(x_ref[...], w_ref[...], preferred_element_type=jnp.float32)
            * dinvb
        )

    return pl.pallas_call(
        body,
        grid=(NP // R,),
        in_specs=[
            pl.BlockSpec((R, D), lambda i: (i, 0)),
            pl.BlockSpec((D, D), lambda i: (0, 0)),
            pl.BlockSpec((NC, R, DW), lambda i: (0, i, 0)),
        ],
        out_specs=[
            pl.BlockSpec((R, D), lambda i: (i, 0)),
            pl.BlockSpec((R, D), lambda i: (i, 0)),
        ],
        out_shape=[
            jax.ShapeDtypeStruct((NP, D), jnp.float32),
            jax.ShapeDtypeStruct((NP, D), jnp.float32),
        ],
    )(x, W, deg_parts)


def _tc_combine_mm(s_parts, hp, dinvb, b2d, W):
    def body(s_ref, hp_ref, dinv_ref, b_ref, w_ref, o_ref):
        dinv = dinv_ref[...]
        h1 = jnp.maximum((s_ref[0] + s_ref[1] + hp_ref[...]) * dinv + b_ref[...], 0.0)
        o_ref[...] = (
            jnp.dot(h1, w_ref[...], preferred_element_type=jnp.float32) * dinv
        )

    return pl.pallas_call(
        body,
        grid=(NP // R,),
        in_specs=[
            pl.BlockSpec((NC, R, D), lambda i: (0, i, 0)),
            pl.BlockSpec((R, D), lambda i: (i, 0)),
            pl.BlockSpec((R, D), lambda i: (i, 0)),
            pl.BlockSpec((1, D), lambda i: (0, 0)),
            pl.BlockSpec((D, D), lambda i: (0, 0)),
        ],
        out_specs=pl.BlockSpec((R, D), lambda i: (i, 0)),
        out_shape=jax.ShapeDtypeStruct((NP, D), jnp.float32),
    )(s_parts, hp, dinvb, b2d, W)


def _tc_combine(s_parts, hp, dinvb, b2d, relu):
    def body(s_ref, hp_ref, dinv_ref, b_ref, o_ref):
        t = (s_ref[0] + s_ref[1] + hp_ref[...]) * dinv_ref[...] + b_ref[...]
        o_ref[...] = jnp.maximum(t, 0.0) if relu else t

    return pl.pallas_call(
        body,
        grid=(NP // R,),
        in_specs=[
            pl.BlockSpec((NC, R, D), lambda i: (0, i, 0)),
            pl.BlockSpec((R, D), lambda i: (i, 0)),
            pl.BlockSpec((R, D), lambda i: (i, 0)),
            pl.BlockSpec((1, D), lambda i: (0, 0)),
        ],
        out_specs=pl.BlockSpec((R, D), lambda i: (i, 0)),
        out_shape=jax.ShapeDtypeStruct((NP, D), jnp.float32),
    )(s_parts, hp, dinvb, b2d)


def kernel(x, edge_index, W1, b1, W2, b2):
    E = edge_index.shape[1]
    n_ch = -(-E // (NW * CH))
    e_pad = NW * n_ch * CH

    pad = N + jnp.arange(e_pad - E, dtype=edge_index.dtype) % (NP - N)
    src3 = jnp.concatenate([edge_index[0], pad]).reshape(NW, n_ch, CH)
    dst3 = jnp.concatenate([edge_index[1], pad]).reshape(NW, n_ch, CH)

    xp = jnp.pad(x, ((0, NP - N), (0, 0)))
    zeros_rows = jnp.zeros((NP, D), jnp.float32)
    zeros_dw = jnp.zeros((NP, DW), jnp.float32)
    ones_dw = jnp.ones((CH, DW), jnp.float32)
    b1_2d = b1.reshape(1, D)
    b2_2d = b2.reshape(1, D)

    deg_parts = _sc_degree(dst3, ones_dw, zeros_dw, n_ch)
    h1p, dinvb = _tc_mm1_scale(xp, W1, deg_parts)

    s1 = _sc_scatter_rows(h1p, src3, dst3, zeros_rows, n_ch)
    h2p = _tc_combine_mm(s1, h1p, dinvb, b1_2d, W2)

    s2 = _sc_scatter_rows(h2p, src3, dst3, zeros_rows, n_ch)
    out = _tc_combine(s2, h2p, dinvb, b2_2d, relu=False)

    return out[:N]


# speedup vs baseline: 27.6219x; 1.0021x over previous
import functools

import jax
import jax.numpy as jnp
from jax import lax
from jax.experimental import pallas as pl
from jax.experimental.pallas import tpu as pltpu
from jax.experimental.pallas import tpu_sc as plsc

N = 10000
D = 128
NP = 10240
NC, NS = 2, 16
NW = NC * NS
CH = 128
R = 1024
DW = 128


def _sc_mesh():
    return plsc.VectorSubcoreMesh(
        core_axis_name="c", subcore_axis_name="s", num_cores=NC, num_subcores=NS
    )


def _sc_scatter_rows(h, src3, dst3, zeros_rows, n_ch):
    rpt = NP // NS

    n_g = (n_ch + 1) // 2

    @functools.partial(
        pl.kernel,
        out_type=jax.ShapeDtypeStruct((NC, NP, D), jnp.float32),
        mesh=_sc_mesh(),
        scratch_types=[
            pltpu.VMEM((n_ch, CH), jnp.int32),
            pltpu.VMEM((CH,), jnp.int32),
            pltpu.VMEM((CH,), jnp.int32),
            pltpu.VMEM((CH, D), jnp.float32),
            pltpu.VMEM((CH, D), jnp.float32),
            pltpu.VMEM_SHARED((NP, D), jnp.float32),
            pltpu.SemaphoreType.DMA,
            pltpu.SemaphoreType.DMA,
        ],
    )
    def k(h_hbm, src_hbm, dst_hbm, z_hbm, out_hbm, src_v, d0, d1, r0, r1, acc,
          sem0, sem1):
        cid = lax.axis_index("c")
        sid = lax.axis_index("s")
        wid = cid * NS + sid
        rows = (r0, r1)
        dsts = (d0, d1)
        sems = (sem0, sem1)
        pltpu.sync_copy(src_hbm.at[wid], src_v)
        pltpu.sync_copy(z_hbm.at[pl.ds(sid * rpt, rpt)], acc.at[pl.ds(sid * rpt, rpt)])
        plsc.subcore_barrier()

        pltpu.async_copy(h_hbm.at[src_v.at[0]], rows[0], sems[0])
        pltpu.async_copy(dst_hbm.at[wid, 0], dsts[0], sems[0])

        def body(g, carry):
            for b in range(2):
                j = g * 2 + b
                nxt = j + 1

                @pl.when(nxt < n_ch)
                def _():
                    pltpu.async_copy(
                        h_hbm.at[src_v.at[nxt]], rows[1 - b], sems[1 - b]
                    )
                    pltpu.async_copy(dst_hbm.at[wid, nxt], dsts[1 - b], sems[1 - b])

                @pl.when(j < n_ch)
                def _():
                    pltpu.make_async_copy(
                        h_hbm.at[src_v.at[j]], rows[b], sems[b]
                    ).wait()
                    pltpu.make_async_copy(
                        dst_hbm.at[wid, j], dsts[b], sems[b]
                    ).wait()
                    pltpu.sync_copy(rows[b], acc.at[dsts[b]], add=True)

            return carry

        lax.fori_loop(0, n_g, body, 0)
        plsc.subcore_barrier()
        pltpu.sync_copy(
            acc.at[pl.ds(sid * rpt, rpt)], out_hbm.at[cid, pl.ds(sid * rpt, rpt)]
        )

    return k(h, src3, dst3, zeros_rows)


def _sc_degree(dst3, ones_rows, zeros_rows, n_ch):
    rpt = NP // NS

    @functools.partial(
        pl.kernel,
        out_type=jax.ShapeDtypeStruct((NC, NP, DW), jnp.float32),
        mesh=_sc_mesh(),
        scratch_types=[
            pltpu.VMEM((n_ch, CH), jnp.int32),
            pltpu.VMEM((CH, DW), jnp.float32),
            pltpu.VMEM_SHARED((NP, DW), jnp.float32),
            pltpu.SemaphoreType.DMA,
        ],
    )
    def k(dst_hbm, ones_hbm, z_hbm, out_hbm, dst_v, ones_v, acc, sem):
        cid = lax.axis_index("c")
        sid = lax.axis_index("s")
        wid = cid * NS + sid
        pltpu.sync_copy(z_hbm.at[pl.ds(sid * rpt, rpt)], acc.at[pl.ds(sid * rpt, rpt)])
        pltpu.sync_copy(dst_hbm.at[wid], dst_v)
        pltpu.sync_copy(ones_hbm, ones_v)
        plsc.subcore_barrier()

        W = 8

        def body(j, carry):
            pltpu.async_copy(ones_v, acc.at[dst_v.at[j]], sem, add=True)

            @pl.when(j >= W)
            def _():
                pltpu.make_async_copy(ones_v, acc.at[dst_v.at[0]], sem).wait()

            return carry

        lax.fori_loop(0, n_ch, body, 0)

        def drain(j, carry):
            pltpu.make_async_copy(ones_v, acc.at[dst_v.at[0]], sem).wait()
            return carry

        lax.fori_loop(0, min(W, n_ch), drain, 0)
        plsc.subcore_barrier()
        pltpu.sync_copy(
            acc.at[pl.ds(sid * rpt, rpt)], out_hbm.at[cid, pl.ds(sid * rpt, rpt)]
        )

    return k(dst3, ones_rows, zeros_rows)


def _tc_mm1_scale(x, W, deg_parts):
    def body(x_ref, w_ref, deg_ref, h1p_ref, dinv_ref):
        d = deg_ref[...]
        degsum = d[0, :, 0:1] + d[1, :, 0:1] + 1.0
        dinvb = jnp.broadcast_to(lax.rsqrt(degsum), (R, D))
        dinv_ref[...] = dinvb
        h1p_ref[...] = (
            jnp.dot(x_ref[...], w_ref[...], preferred_element_type=jnp.float32)
            * dinvb
        )

    return pl.pallas_call(
        body,
        grid=(NP // R,),
        in_specs=[
            pl.BlockSpec((R, D), lambda i: (i, 0)),
            pl.BlockSpec((D, D), lambda i: (0, 0)),
            pl.BlockSpec((NC, R, DW), lambda i: (0, i, 0)),
        ],
        out_specs=[
            pl.BlockSpec((R, D), lambda i: (i, 0)),
            pl.BlockSpec((R, D), lambda i: (i, 0)),
        ],
        out_shape=[
            jax.ShapeDtypeStruct((NP, D), jnp.float32),
            jax.ShapeDtypeStruct((NP, D), jnp.float32),
        ],
    )(x, W, deg_parts)


def _tc_combine_mm(s_parts, hp, dinvb, b2d, W):
    def body(s_ref, hp_ref, dinv_ref, b_ref, w_ref, o_ref):
        dinv = dinv_ref[...]
        h1 = jnp.maximum((s_ref[0] + s_ref[1] + hp_ref[...]) * dinv + b_ref[...], 0.0)
        o_ref[...] = (
            jnp.dot(h1, w_ref[...], preferred_element_type=jnp.float32) * dinv
        )

    return pl.pallas_call(
        body,
        grid=(NP // R,),
        in_specs=[
            pl.BlockSpec((NC, R, D), lambda i: (0, i, 0)),
            pl.BlockSpec((R, D), lambda i: (i, 0)),
            pl.BlockSpec((R, D), lambda i: (i, 0)),
            pl.BlockSpec((1, D), lambda i: (0, 0)),
            pl.BlockSpec((D, D), lambda i: (0, 0)),
        ],
        out_specs=pl.BlockSpec((R, D), lambda i: (i, 0)),
        out_shape=jax.ShapeDtypeStruct((NP, D), jnp.float32),
    )(s_parts, hp, dinvb, b2d, W)


def _tc_combine(s_parts, hp, dinvb, b2d, relu):
    def body(s_ref, hp_ref, dinv_ref, b_ref, o_ref):
        t = (s_ref[0] + s_ref[1] + hp_ref[...]) * dinv_ref[...] + b_ref[...]
        o_ref[...] = jnp.maximum(t, 0.0) if relu else t

    return pl.pallas_call(
        body,
        grid=(NP // R,),
        in_specs=[
            pl.BlockSpec((NC, R, D), lambda i: (0, i, 0)),
            pl.BlockSpec((R, D), lambda i: (i, 0)),
            pl.BlockSpec((R, D), lambda i: (i, 0)),
            pl.BlockSpec((1, D), lambda i: (0, 0)),
        ],
        out_specs=pl.BlockSpec((R, D), lambda i: (i, 0)),
        out_shape=jax.ShapeDtypeStruct((NP, D), jnp.float32),
    )(s_parts, hp, dinvb, b2d)


def kernel(x, edge_index, W1, b1, W2, b2):
    E = edge_index.shape[1]
    n_ch = -(-E // (NW * CH))
    e_pad = NW * n_ch * CH

    pad = N + jnp.arange(e_pad - E, dtype=edge_index.dtype) % (NP - N)
    src3 = jnp.concatenate([edge_index[0], pad]).reshape(NW, n_ch, CH)
    dst3 = jnp.concatenate([edge_index[1], pad]).reshape(NW, n_ch, CH)

    xp = jnp.pad(x, ((0, NP - N), (0, 0)))
    zeros_rows = jnp.zeros((NP, D), jnp.float32)
    zeros_dw = jnp.zeros((NP, DW), jnp.float32)
    ones_dw = jnp.ones((CH, DW), jnp.float32)
    b1_2d = b1.reshape(1, D)
    b2_2d = b2.reshape(1, D)

    deg_parts = _sc_degree(dst3, ones_dw, zeros_dw, n_ch)
    h1p, dinvb = _tc_mm1_scale(xp, W1, deg_parts)

    s1 = _sc_scatter_rows(h1p, src3, dst3, zeros_rows, n_ch)
    h2p = _tc_combine_mm(s1, h1p, dinvb, b1_2d, W2)

    s2 = _sc_scatter_rows(h2p, src3, dst3, zeros_rows, n_ch)
    out = _tc_combine(s2, h2p, dinvb, b2_2d, relu=False)

    return out[:N]


# speedup vs baseline: 27.9341x; 1.0113x over previous
import functools

import jax
import jax.numpy as jnp
from jax import lax
from jax.experimental import pallas as pl
from jax.experimental.pallas import tpu as pltpu
from jax.experimental.pallas import tpu_sc as plsc

N = 10000
D = 128
NP = 10240
NC, NS = 2, 16
NW = NC * NS
CH = 128
R = 1024
DW = 128


def _sc_mesh():
    return plsc.VectorSubcoreMesh(
        core_axis_name="c", subcore_axis_name="s", num_cores=NC, num_subcores=NS
    )


def _sc_scatter_rows(h, src3, dst3, zeros_rows, n_ch):
    rpt = NP // NS

    n_g = (n_ch + 1) // 2

    @functools.partial(
        pl.kernel,
        out_type=jax.ShapeDtypeStruct((NC, NP, D), jnp.float32),
        mesh=_sc_mesh(),
        scratch_types=[
            pltpu.VMEM((n_ch, CH), jnp.int32),
            pltpu.VMEM((CH,), jnp.int32),
            pltpu.VMEM((CH,), jnp.int32),
            pltpu.VMEM((CH, D), jnp.float32),
            pltpu.VMEM((CH, D), jnp.float32),
            pltpu.VMEM_SHARED((NP, D), jnp.float32),
            pltpu.SemaphoreType.DMA,
            pltpu.SemaphoreType.DMA,
        ],
    )
    def k(h_hbm, src_hbm, dst_hbm, z_hbm, out_hbm, src_v, d0, d1, r0, r1, acc,
          sem0, sem1):
        cid = lax.axis_index("c")
        sid = lax.axis_index("s")
        wid = cid * NS + sid
        rows = (r0, r1)
        dsts = (d0, d1)
        sems = (sem0, sem1)
        pltpu.sync_copy(src_hbm.at[wid], src_v)
        pltpu.sync_copy(z_hbm.at[pl.ds(sid * rpt, rpt)], acc.at[pl.ds(sid * rpt, rpt)])
        plsc.subcore_barrier()

        pltpu.async_copy(h_hbm.at[src_v.at[0]], rows[0], sems[0])
        pltpu.async_copy(dst_hbm.at[wid, 0], dsts[0], sems[0])

        def body(g, carry):
            for b in range(2):
                j = g * 2 + b
                nxt = j + 1

                @pl.when(nxt < n_ch)
                def _():
                    pltpu.async_copy(
                        h_hbm.at[src_v.at[nxt]], rows[1 - b], sems[1 - b]
                    )
                    pltpu.async_copy(dst_hbm.at[wid, nxt], dsts[1 - b], sems[1 - b])

                @pl.when(j < n_ch)
                def _():
                    pltpu.make_async_copy(
                        h_hbm.at[src_v.at[j]], rows[b], sems[b]
                    ).wait()
                    pltpu.make_async_copy(
                        dst_hbm.at[wid, j], dsts[b], sems[b]
                    ).wait()
                    pltpu.sync_copy(rows[b], acc.at[dsts[b]], add=True)

            return carry

        lax.fori_loop(0, n_g, body, 0)
        plsc.subcore_barrier()
        pltpu.sync_copy(
            acc.at[pl.ds(sid * rpt, rpt)], out_hbm.at[cid, pl.ds(sid * rpt, rpt)]
        )

    return k(h, src3, dst3, zeros_rows)


def _sc_degree(dst3, ones_rows, zeros_rows, n_ch):
    rpt = NP // NS

    @functools.partial(
        pl.kernel,
        out_type=jax.ShapeDtypeStruct((NC, NP, DW), jnp.float32),
        mesh=_sc_mesh(),
        scratch_types=[
            pltpu.VMEM((n_ch, CH), jnp.int32),
            pltpu.VMEM((CH, DW), jnp.float32),
            pltpu.VMEM_SHARED((NP, DW), jnp.float32),
            pltpu.SemaphoreType.DMA,
        ],
    )
    def k(dst_hbm, ones_hbm, z_hbm, out_hbm, dst_v, ones_v, acc, sem):
        cid = lax.axis_index("c")
        sid = lax.axis_index("s")
        wid = cid * NS + sid
        pltpu.sync_copy(z_hbm.at[pl.ds(sid * rpt, rpt)], acc.at[pl.ds(sid * rpt, rpt)])
        pltpu.sync_copy(dst_hbm.at[wid], dst_v)
        pltpu.sync_copy(ones_hbm, ones_v)
        plsc.subcore_barrier()

        W = 8

        def body(j, carry):
            pltpu.async_copy(ones_v, acc.at[dst_v.at[j]], sem, add=True)

            @pl.when(j >= W)
            def _():
                pltpu.make_async_copy(ones_v, acc.at[dst_v.at[0]], sem).wait()

            return carry

        lax.fori_loop(0, n_ch, body, 0)

        def drain(j, carry):
            pltpu.make_async_copy(ones_v, acc.at[dst_v.at[0]], sem).wait()
            return carry

        lax.fori_loop(0, min(W, n_ch), drain, 0)
        plsc.subcore_barrier()
        pltpu.sync_copy(
            acc.at[pl.ds(sid * rpt, rpt)], out_hbm.at[cid, pl.ds(sid * rpt, rpt)]
        )

    return k(dst3, ones_rows, zeros_rows)


def _tc_matmul(x, W):
    def body(x_ref, w_ref, o_ref):
        o_ref[...] = jnp.dot(x_ref[...], w_ref[...], preferred_element_type=jnp.float32)

    return pl.pallas_call(
        body,
        grid=(NP // R,),
        in_specs=[
            pl.BlockSpec((R, D), lambda i: (i, 0)),
            pl.BlockSpec((D, D), lambda i: (0, 0)),
        ],
        out_specs=pl.BlockSpec((R, D), lambda i: (i, 0)),
        out_shape=jax.ShapeDtypeStruct((NP, D), jnp.float32),
    )(x, W)


def _tc_scale(deg_parts, h_raw):
    def body(deg_ref, h_ref, h1p_ref, dinv_ref):
        d = deg_ref[...]
        degsum = d[0, :, 0:1] + d[1, :, 0:1] + 1.0
        dinvb = jnp.broadcast_to(lax.rsqrt(degsum), (R, D))
        dinv_ref[...] = dinvb
        h1p_ref[...] = h_ref[...] * dinvb

    return pl.pallas_call(
        body,
        grid=(NP // R,),
        in_specs=[
            pl.BlockSpec((NC, R, DW), lambda i: (0, i, 0)),
            pl.BlockSpec((R, D), lambda i: (i, 0)),
        ],
        out_specs=[
            pl.BlockSpec((R, D), lambda i: (i, 0)),
            pl.BlockSpec((R, D), lambda i: (i, 0)),
        ],
        out_shape=[
            jax.ShapeDtypeStruct((NP, D), jnp.float32),
            jax.ShapeDtypeStruct((NP, D), jnp.float32),
        ],
    )(deg_parts, h_raw)


def _tc_combine_mm(s_parts, hp, dinvb, b2d, W):
    def body(s_ref, hp_ref, dinv_ref, b_ref, w_ref, o_ref):
        dinv = dinv_ref[...]
        h1 = jnp.maximum((s_ref[0] + s_ref[1] + hp_ref[...]) * dinv + b_ref[...], 0.0)
        o_ref[...] = (
            jnp.dot(h1, w_ref[...], preferred_element_type=jnp.float32) * dinv
        )

    return pl.pallas_call(
        body,
        grid=(NP // R,),
        in_specs=[
            pl.BlockSpec((NC, R, D), lambda i: (0, i, 0)),
            pl.BlockSpec((R, D), lambda i: (i, 0)),
            pl.BlockSpec((R, D), lambda i: (i, 0)),
            pl.BlockSpec((1, D), lambda i: (0, 0)),
            pl.BlockSpec((D, D), lambda i: (0, 0)),
        ],
        out_specs=pl.BlockSpec((R, D), lambda i: (i, 0)),
        out_shape=jax.ShapeDtypeStruct((NP, D), jnp.float32),
    )(s_parts, hp, dinvb, b2d, W)


def _tc_combine(s_parts, hp, dinvb, b2d, relu):
    def body(s_ref, hp_ref, dinv_ref, b_ref, o_ref):
        t = (s_ref[0] + s_ref[1] + hp_ref[...]) * dinv_ref[...] + b_ref[...]
        o_ref[...] = jnp.maximum(t, 0.0) if relu else t

    return pl.pallas_call(
        body,
        grid=(NP // R,),
        in_specs=[
            pl.BlockSpec((NC, R, D), lambda i: (0, i, 0)),
            pl.BlockSpec((R, D), lambda i: (i, 0)),
            pl.BlockSpec((R, D), lambda i: (i, 0)),
            pl.BlockSpec((1, D), lambda i: (0, 0)),
        ],
        out_specs=pl.BlockSpec((R, D), lambda i: (i, 0)),
        out_shape=jax.ShapeDtypeStruct((N, D), jnp.float32),
    )(s_parts, hp, dinvb, b2d)


def kernel(x, edge_index, W1, b1, W2, b2):
    E = edge_index.shape[1]
    n_ch = -(-E // (NW * CH))
    e_pad = NW * n_ch * CH

    pad = N + jnp.arange(e_pad - E, dtype=edge_index.dtype) % (NP - N)
    src3 = jnp.concatenate([edge_index[0], pad]).reshape(NW, n_ch, CH)
    dst3 = jnp.concatenate([edge_index[1], pad]).reshape(NW, n_ch, CH)

    xp = jnp.pad(x, ((0, NP - N), (0, 0)))
    zeros_rows = jnp.zeros((NP, D), jnp.float32)
    zeros_dw = jnp.zeros((NP, DW), jnp.float32)
    ones_dw = jnp.ones((CH, DW), jnp.float32)
    b1_2d = b1.reshape(1, D)
    b2_2d = b2.reshape(1, D)

    h_raw = _tc_matmul(xp, W1)
    deg_parts = _sc_degree(dst3, ones_dw, zeros_dw, n_ch)
    h1p, dinvb = _tc_scale(deg_parts, h_raw)

    s1 = _sc_scatter_rows(h1p, src3, dst3, zeros_rows, n_ch)
    h2p = _tc_combine_mm(s1, h1p, dinvb, b1_2d, W2)

    s2 = _sc_scatter_rows(h2p, src3, dst3, zeros_rows, n_ch)
    return _tc_combine(s2, h2p, dinvb, b2_2d, relu=False)
